# baseline (speedup 1.0000x reference)
import functools

import jax
import jax.numpy as jnp
from jax import lax
from jax.experimental import pallas as pl
from jax.experimental.pallas import tpu as pltpu
from jax.experimental.pallas import tpu_sc as plsc

_F = 14
_B = 16384
_D = 32
_NC = 2
_NS = 16
_NW = _NC * _NS
_BPW = _B // _NW
_CH = 128
_NCH = _BPW // _CH

_COL = tuple(range(_F))

_mesh = plsc.VectorSubcoreMesh(core_axis_name="c", subcore_axis_name="s")


@functools.partial(
    pl.kernel,
    mesh=_mesh,
    out_type=jax.ShapeDtypeStruct((_B, _F * _D), jnp.float32),
    scratch_types=(
        [pltpu.VMEM((_BPW,), jnp.int32) for _ in range(_F)]
        + [pltpu.VMEM((2, _CH, _D), jnp.float32) for _ in range(_F)]
        + [pltpu.SemaphoreType.DMA, pltpu.SemaphoreType.DMA]
    ),
    compiler_params=pltpu.CompilerParams(use_tc_tiling_on_sc=False),
)
def _sc_gather(*refs):
    big = refs[0]
    tbls = [big.at[f] for f in range(_F)]
    idxs = refs[1:_F + 1]
    out = refs[_F + 1]
    idx_vs = refs[_F + 2:2 * _F + 2]
    rows_vs = refs[2 * _F + 2:3 * _F + 2]
    gsem, wsem = refs[3 * _F + 2:]

    wid = lax.axis_index("s") * _NC + lax.axis_index("c")
    base = wid * _BPW

    for f in range(_F):
        pltpu.sync_copy(idxs[f].at[pl.ds(base, _BPW)], idx_vs[f])

    writes = {0: [], 1: []}
    for c in range(_NCH):
        p = c % 2
        for h in writes[p]:
            h.wait()
        gathers = [
            pltpu.async_copy(
                tbls[f].at[idx_vs[f].at[pl.ds(c * _CH, _CH)]],
                rows_vs[f].at[p],
                gsem,
            )
            for f in range(_F)
        ]
        for h in gathers:
            h.wait()
        writes[p] = [
            pltpu.async_copy(
                rows_vs[f].at[p],
                out.at[pl.ds(base + c * _CH, _CH), pl.ds(_COL[f] * _D, _D)],
                wsem,
            )
            for f in range(_F)
        ]
    for p in (0, 1):
        for h in writes[p]:
            h.wait()

    return None


def kernel(idx_story_id, tbl_story_id, idx_story_title, tbl_story_title, idx_source_id, tbl_source_id, idx_author_id, tbl_author_id, idx_most_frequent_keyword, tbl_most_frequent_keyword, idx_most_frequent_entity, tbl_most_frequent_entity, idx_source_alexa_rank, tbl_source_alexa_rank, idx_read_count, tbl_read_count, idx_shared_count, tbl_shared_count, idx_angry_count, tbl_angry_count, idx_cry_count, tbl_cry_count, idx_neutral_count, tbl_neutral_count, idx_smile_count, tbl_smile_count, idx_happy_count, tbl_happy_count):
    tables = (tbl_story_id, tbl_story_title, tbl_source_id, tbl_author_id,
              tbl_most_frequent_keyword, tbl_most_frequent_entity,
              tbl_source_alexa_rank, tbl_read_count, tbl_shared_count,
              tbl_angry_count, tbl_cry_count, tbl_neutral_count,
              tbl_smile_count, tbl_happy_count)
    indices = (idx_story_id, idx_story_title, idx_source_id, idx_author_id,
               idx_most_frequent_keyword, idx_most_frequent_entity,
               idx_source_alexa_rank, idx_read_count, idx_shared_count,
               idx_angry_count, idx_cry_count, idx_neutral_count,
               idx_smile_count, idx_happy_count)
    big = jnp.stack(tables)
    return _sc_gather(big, *indices)


# speedup vs baseline: 4.3218x; 4.3218x over previous
import functools

import jax
import jax.numpy as jnp
from jax import lax
from jax.experimental import pallas as pl
from jax.experimental.pallas import tpu as pltpu
from jax.experimental.pallas import tpu_sc as plsc

_F = 14
_B = 16384
_D = 32
_NC = 2
_NS = 16
_NW = _NC * _NS
_BPW = _B // _NW
_CH = 128
_NCH = _BPW // _CH

_COL = tuple(_F - 1 - f for f in range(_F))

_mesh = plsc.VectorSubcoreMesh(core_axis_name="c", subcore_axis_name="s")


@functools.partial(
    pl.kernel,
    mesh=_mesh,
    out_type=jax.ShapeDtypeStruct((_B, _F * _D), jnp.float32),
    scratch_types=(
        [pltpu.VMEM((_BPW,), jnp.int32) for _ in range(_F)]
        + [pltpu.VMEM((2, _CH, _D), jnp.float32) for _ in range(_F)]
        + [pltpu.SemaphoreType.DMA, pltpu.SemaphoreType.DMA]
    ),
    compiler_params=pltpu.CompilerParams(use_tc_tiling_on_sc=False),
)
def _sc_gather(*refs):
    tbls = refs[:_F]
    idxs = refs[_F:2 * _F]
    out = refs[2 * _F]
    idx_vs = refs[2 * _F + 1:3 * _F + 1]
    rows_vs = refs[3 * _F + 1:4 * _F + 1]
    gsem, wsem = refs[4 * _F + 1:]

    wid = lax.axis_index("s") * _NC + lax.axis_index("c")
    base = wid * _BPW

    for f in range(_F):
        pltpu.sync_copy(idxs[f].at[pl.ds(base, _BPW)], idx_vs[f])

    writes = {0: [], 1: []}
    for c in range(_NCH):
        p = c % 2
        for h in writes[p]:
            h.wait()
        gathers = [
            pltpu.async_copy(
                tbls[f].at[idx_vs[f].at[pl.ds(c * _CH, _CH)]],
                rows_vs[f].at[p],
                gsem,
            )
            for f in range(_F)
        ]
        for h in gathers:
            h.wait()
        writes[p] = [
            pltpu.async_copy(
                rows_vs[f].at[p],
                out.at[pl.ds(base + c * _CH, _CH), pl.ds(_COL[f] * _D, _D)],
                wsem,
            )
            for f in range(_F)
        ]
    for p in (0, 1):
        for h in writes[p]:
            h.wait()

    return None


def kernel(idx_story_id, tbl_story_id, idx_story_title, tbl_story_title, idx_source_id, tbl_source_id, idx_author_id, tbl_author_id, idx_most_frequent_keyword, tbl_most_frequent_keyword, idx_most_frequent_entity, tbl_most_frequent_entity, idx_source_alexa_rank, tbl_source_alexa_rank, idx_read_count, tbl_read_count, idx_shared_count, tbl_shared_count, idx_angry_count, tbl_angry_count, idx_cry_count, tbl_cry_count, idx_neutral_count, tbl_neutral_count, idx_smile_count, tbl_smile_count, idx_happy_count, tbl_happy_count):
    tables = (tbl_story_id, tbl_story_title, tbl_source_id, tbl_author_id,
              tbl_most_frequent_keyword, tbl_most_frequent_entity,
              tbl_source_alexa_rank, tbl_read_count, tbl_shared_count,
              tbl_angry_count, tbl_cry_count, tbl_neutral_count,
              tbl_smile_count, tbl_happy_count)
    indices = (idx_story_id, idx_story_title, idx_source_id, idx_author_id,
               idx_most_frequent_keyword, idx_most_frequent_entity,
               idx_source_alexa_rank, idx_read_count, idx_shared_count,
               idx_angry_count, idx_cry_count, idx_neutral_count,
               idx_smile_count, idx_happy_count)
    rev = tuple(reversed(range(_F)))
    return _sc_gather(*(tables[i] for i in rev), *(indices[i] for i in rev))


# speedup vs baseline: 4.3235x; 1.0004x over previous
import functools

import jax
import jax.numpy as jnp
from jax import lax
from jax.experimental import pallas as pl
from jax.experimental.pallas import tpu as pltpu
from jax.experimental.pallas import tpu_sc as plsc

_F = 14
_B = 16384
_D = 32
_NC = 2
_NS = 16
_NW = _NC * _NS
_BPW = _B // _NW
_CH = 128
_NCH = _BPW // _CH

_mesh = plsc.VectorSubcoreMesh(core_axis_name="c", subcore_axis_name="s")


@functools.partial(
    pl.kernel,
    mesh=_mesh,
    out_type=jax.ShapeDtypeStruct((_B, _F * _D), jnp.float32),
    scratch_types=(
        [pltpu.VMEM((_BPW,), jnp.int32) for _ in range(_F)]
        + [pltpu.VMEM((2, _CH, _D), jnp.float32) for _ in range(_F)]
        + [pltpu.SemaphoreType.DMA, pltpu.SemaphoreType.DMA]
    ),
    compiler_params=pltpu.CompilerParams(use_tc_tiling_on_sc=False),
)
def _sc_gather(*refs):
    tbls = refs[:_F]
    idxs = refs[_F:2 * _F]
    out = refs[2 * _F]
    idx_vs = refs[2 * _F + 1:3 * _F + 1]
    rows_vs = refs[3 * _F + 1:4 * _F + 1]
    gsem, wsem = refs[4 * _F + 1:]

    wid = lax.axis_index("s") * _NC + lax.axis_index("c")
    base = wid * _BPW

    for f in range(_F):
        pltpu.sync_copy(idxs[f].at[pl.ds(base, _BPW)], idx_vs[f])

    writes = {0: [], 1: []}
    for c in range(_NCH):
        p = c % 2
        for h in writes[p]:
            h.wait()
        gathers = [
            pltpu.async_copy(
                tbls[f].at[idx_vs[f].at[pl.ds(c * _CH, _CH)]],
                rows_vs[f].at[p],
                gsem,
            )
            for f in range(_F)
        ]
        for h in gathers:
            h.wait()
        writes[p] = [
            pltpu.async_copy(
                rows_vs[f].at[p],
                out.at[pl.ds(base + c * _CH, _CH), pl.ds(f * _D, _D)],
                wsem,
            )
            for f in range(_F)
        ]
    for p in (0, 1):
        for h in writes[p]:
            h.wait()

    return None


def kernel(idx_story_id, tbl_story_id, idx_story_title, tbl_story_title, idx_source_id, tbl_source_id, idx_author_id, tbl_author_id, idx_most_frequent_keyword, tbl_most_frequent_keyword, idx_most_frequent_entity, tbl_most_frequent_entity, idx_source_alexa_rank, tbl_source_alexa_rank, idx_read_count, tbl_read_count, idx_shared_count, tbl_shared_count, idx_angry_count, tbl_angry_count, idx_cry_count, tbl_cry_count, idx_neutral_count, tbl_neutral_count, idx_smile_count, tbl_smile_count, idx_happy_count, tbl_happy_count):
    tables = (tbl_story_id, tbl_story_title, tbl_source_id, tbl_author_id,
              tbl_most_frequent_keyword, tbl_most_frequent_entity,
              tbl_source_alexa_rank, tbl_read_count, tbl_shared_count,
              tbl_angry_count, tbl_cry_count, tbl_neutral_count,
              tbl_smile_count, tbl_happy_count)
    indices = (idx_story_id, idx_story_title, idx_source_id, idx_author_id,
               idx_most_frequent_keyword, idx_most_frequent_entity,
               idx_source_alexa_rank, idx_read_count, idx_shared_count,
               idx_angry_count, idx_cry_count, idx_neutral_count,
               idx_smile_count, idx_happy_count)
    return _sc_gather(*tables, *indices)
